# baseline (speedup 1.0000x reference)
import jax
import jax.numpy as jnp
import numpy as np
from jax.experimental import pallas as pl
from jax.experimental.pallas import tpu as pltpu

B, S, H = 128, 8192, 128
BT = 8
C = 1024
NC = S // C
NEG = float(np.log(np.float32(1e-45)))


def _dot_t(a, b):
    return jax.lax.dot_general(a.astype(jnp.bfloat16), b.astype(jnp.bfloat16),
                               (((1,), (1,)), ((), ())),
                               preferred_element_type=jnp.float32)


def _attn_kernel(mask_ref, user_ref, seq_ref, wenc_ref, benc_ref,
                 w1_ref, w2_ref, vt_ref,
                 out_p_ref, out_i_ref, m_s, se_s, ix_s):
    f32 = jnp.float32
    c = pl.program_id(1)

    @pl.when(c == 0)
    def _init():
        m_s[...] = jnp.full((BT, 1), -jnp.inf, f32)
        se_s[...] = jnp.zeros((BT, 1), f32)
        ix_s[...] = jnp.zeros((BT, 1), jnp.int32)

    iota = jax.lax.broadcasted_iota(jnp.int32, (C, 1), 0)
    off = c * C

    for i in range(BT):
        enc = _dot_t(seq_ref[i, 0], wenc_ref[...]) + benc_ref[...]
        et = _dot_t(enc, w1_ref[...])
        dt = _dot_t(user_ref[i], w2_ref[...])
        u = _dot_t(jnp.tanh(et + dt), vt_ref[...])[:, 0:1]

        score = jnp.where(mask_ref[i, 0], u, f32(NEG)) * f32(10.0)

        m_c = jnp.max(score, axis=0, keepdims=True)
        se_c = jnp.sum(jnp.exp(score - m_c), axis=0, keepdims=True)
        ix_c = jnp.min(jnp.where(score == m_c, iota, jnp.int32(C)),
                       axis=0, keepdims=True) + off

        m_old = m_s[i:i + 1, :]
        se_old = se_s[i:i + 1, :]
        ix_old = ix_s[i:i + 1, :]
        m_new = jnp.maximum(m_old, m_c)
        m_s[i:i + 1, :] = m_new
        se_s[i:i + 1, :] = (se_old * jnp.exp(m_old - m_new)
                            + se_c * jnp.exp(m_c - m_new))
        ix_s[i:i + 1, :] = jnp.where(m_c > m_old, ix_c, ix_old)

    @pl.when(c == NC - 1)
    def _fin():
        out_p_ref[...] = f32(1.0) / se_s[...]
        out_i_ref[...] = ix_s[...]


@jax.jit
def kernel(mask, user, static_server_seq, tmp_server_capacity, server_active,
           Wenc, benc, W1, W2, vt):
    seq = jnp.concatenate(
        [static_server_seq, tmp_server_capacity, server_active], axis=-1)
    seq4 = seq.reshape(B, NC, C, 4)
    mask4 = mask.reshape(B, NC, C, 1)
    benc2 = benc.reshape(1, H)
    vt8 = jnp.broadcast_to(vt, (8, H))

    seqm = lambda bt, c: (bt, c, 0, 0)
    rowc = lambda bt, c: (bt, 0, 0)
    rep2 = lambda bt, c: (0, 0)

    top_p, top_i = pl.pallas_call(
        _attn_kernel,
        grid=(B // BT, NC),
        in_specs=[
            pl.BlockSpec((BT, 1, C, 1), seqm),
            pl.BlockSpec((BT, 1, H), rowc),
            pl.BlockSpec((BT, 1, C, 4), seqm),
            pl.BlockSpec((H, 4), rep2),
            pl.BlockSpec((1, H), rep2),
            pl.BlockSpec((H, H), rep2),
            pl.BlockSpec((H, H), rep2),
            pl.BlockSpec((8, H), rep2),
        ],
        out_specs=[
            pl.BlockSpec((BT, 1), lambda bt, c: (bt, 0)),
            pl.BlockSpec((BT, 1), lambda bt, c: (bt, 0)),
        ],
        out_shape=[
            jax.ShapeDtypeStruct((B, 1), jnp.float32),
            jax.ShapeDtypeStruct((B, 1), jnp.int32),
        ],
        scratch_shapes=[
            pltpu.VMEM((BT, 1), jnp.float32),
            pltpu.VMEM((BT, 1), jnp.float32),
            pltpu.VMEM((BT, 1), jnp.int32),
        ],
    )(mask4, user, seq4, Wenc, benc2, W1, W2, vt8)

    return (top_p.reshape(B), top_i.reshape(B))


# speedup vs baseline: 1.2641x; 1.2641x over previous
import jax
import jax.numpy as jnp
import numpy as np
from jax.experimental import pallas as pl

B, S, H = 128, 8192, 128
NSUB = 8
CS = S // NSUB
NEG = float(np.log(np.float32(1e-45)))


def _dot_t(a, b):
    return jax.lax.dot_general(a.astype(jnp.bfloat16), b.astype(jnp.bfloat16),
                               (((1,), (1,)), ((), ())),
                               preferred_element_type=jnp.float32)


def _attn_kernel(mask_ref, user_ref, seq_ref, wenc_ref, benc_ref,
                 w1_ref, w2_ref, vt_ref, out_p_ref, out_i_ref):
    f32 = jnp.float32
    b = pl.program_id(0)

    dt = _dot_t(user_ref[0], w2_ref[...])
    wenc = wenc_ref[...]
    benc = benc_ref[...]
    w1 = w1_ref[...]
    vt8 = vt_ref[...]

    us = []
    for j in range(NSUB):
        seq_j = seq_ref[j * CS:(j + 1) * CS, :]
        enc = _dot_t(seq_j, wenc) + benc
        et = _dot_t(enc, w1)
        t = jnp.tanh(et + dt)
        us.append(_dot_t(t, vt8)[:, 0:1])

    u = jnp.concatenate(us, axis=0)
    u2 = u.reshape(S // 128, 128)

    score = jnp.where(mask_ref[...], u2, f32(NEG)) * f32(10.0)

    m = jnp.max(score)
    se = jnp.sum(jnp.exp(score - m))
    iota = (jax.lax.broadcasted_iota(jnp.int32, (S // 128, 128), 0) * 128
            + jax.lax.broadcasted_iota(jnp.int32, (S // 128, 128), 1))
    ix = jnp.min(jnp.where(score == m, iota, jnp.int32(S)))

    rowsel = jax.lax.broadcasted_iota(jnp.int32, (8, 1), 0) == (b % 8)
    out_p_ref[...] = jnp.where(rowsel, f32(1.0) / se, out_p_ref[...])
    out_i_ref[...] = jnp.where(rowsel, ix, out_i_ref[...])


@jax.jit
def kernel(mask, user, static_server_seq, tmp_server_capacity, server_active,
           Wenc, benc, W1, W2, vt):
    seq = jnp.concatenate(
        [static_server_seq, tmp_server_capacity, server_active],
        axis=-1).reshape(B * S, 4)
    mask2 = mask.reshape(B * S // 128, 128)
    benc2 = benc.reshape(1, H)
    vt8 = jnp.broadcast_to(vt, (8, H))

    rep2 = lambda b: (0, 0)

    top_p, top_i = pl.pallas_call(
        _attn_kernel,
        grid=(B,),
        in_specs=[
            pl.BlockSpec((S // 128, 128), lambda b: (b, 0)),
            pl.BlockSpec((1, 1, H), lambda b: (b, 0, 0)),
            pl.BlockSpec((S, 4), lambda b: (b, 0)),
            pl.BlockSpec((H, 4), rep2),
            pl.BlockSpec((1, H), rep2),
            pl.BlockSpec((H, H), rep2),
            pl.BlockSpec((H, H), rep2),
            pl.BlockSpec((8, H), rep2),
        ],
        out_specs=[
            pl.BlockSpec((8, 1), lambda b: (b // 8, 0)),
            pl.BlockSpec((8, 1), lambda b: (b // 8, 0)),
        ],
        out_shape=[
            jax.ShapeDtypeStruct((B, 1), jnp.float32),
            jax.ShapeDtypeStruct((B, 1), jnp.int32),
        ],
    )(mask2, user, seq, Wenc, benc2, W1, W2, vt8)

    return (top_p.reshape(B), top_i.reshape(B))


# speedup vs baseline: 1.4139x; 1.1185x over previous
import jax
import jax.numpy as jnp
import numpy as np
from jax.experimental import pallas as pl

B, S, H = 128, 8192, 128
NSUB = 8
CS = S // NSUB
NEG = float(np.log(np.float32(1e-45)))


def _dot_t(a, b):
    return jax.lax.dot_general(a.astype(jnp.bfloat16), b.astype(jnp.bfloat16),
                               (((1,), (1,)), ((), ())),
                               preferred_element_type=jnp.float32)


def _attn_kernel(mask_ref, user_ref, st_ref, cap_ref, act_ref,
                 wenc_ref, benc_ref, w1_ref, w2_ref, vt_ref,
                 out_p_ref, out_i_ref):
    f32 = jnp.float32
    b = pl.program_id(0)

    dt = _dot_t(user_ref[0], w2_ref[...])
    wenc = wenc_ref[...]
    benc = benc_ref[...]
    w1 = w1_ref[...]
    vt8 = vt_ref[...]

    us = []
    for j in range(NSUB):
        sl = slice(j * CS, (j + 1) * CS)
        seq_j = jnp.concatenate(
            [st_ref[0, sl, :], cap_ref[0, sl, :], act_ref[0, sl, :]],
            axis=1)
        enc = _dot_t(seq_j, wenc) + benc
        et = _dot_t(enc, w1)
        t = jnp.tanh(et + dt)
        us.append(_dot_t(t, vt8)[:, 0:1])

    u = jnp.concatenate(us, axis=0)
    u2 = u.reshape(S // 128, 128)

    score = jnp.where(mask_ref[0] != 0, u2, f32(NEG)) * f32(10.0)

    m = jnp.max(score)
    se = jnp.sum(jnp.exp(score - m))
    iota = (jax.lax.broadcasted_iota(jnp.int32, (S // 128, 128), 0) * 128
            + jax.lax.broadcasted_iota(jnp.int32, (S // 128, 128), 1))
    ix = jnp.min(jnp.where(score == m, iota, jnp.int32(S)))

    rowsel = jax.lax.broadcasted_iota(jnp.int32, (8, 1), 0) == (b % 8)
    out_p_ref[...] = jnp.where(rowsel, f32(1.0) / se, out_p_ref[...])
    out_i_ref[...] = jnp.where(rowsel, ix, out_i_ref[...])


@jax.jit
def kernel(mask, user, static_server_seq, tmp_server_capacity, server_active,
           Wenc, benc, W1, W2, vt):
    mask_f = mask.reshape(B, S // 128, 128).astype(jnp.float32)
    benc2 = benc.reshape(1, H)
    vt8 = jnp.broadcast_to(vt, (8, H))

    row3 = lambda b: (b, 0, 0)
    rep2 = lambda b: (0, 0)

    top_p, top_i = pl.pallas_call(
        _attn_kernel,
        grid=(B,),
        in_specs=[
            pl.BlockSpec((1, S // 128, 128), row3),
            pl.BlockSpec((1, 1, H), row3),
            pl.BlockSpec((1, S, 2), row3),
            pl.BlockSpec((1, S, 1), row3),
            pl.BlockSpec((1, S, 1), row3),
            pl.BlockSpec((H, 4), rep2),
            pl.BlockSpec((1, H), rep2),
            pl.BlockSpec((H, H), rep2),
            pl.BlockSpec((H, H), rep2),
            pl.BlockSpec((8, H), rep2),
        ],
        out_specs=[
            pl.BlockSpec((8, 1), lambda b: (b // 8, 0)),
            pl.BlockSpec((8, 1), lambda b: (b // 8, 0)),
        ],
        out_shape=[
            jax.ShapeDtypeStruct((B, 1), jnp.float32),
            jax.ShapeDtypeStruct((B, 1), jnp.int32),
        ],
    )(mask_f, user, static_server_seq, tmp_server_capacity, server_active,
      Wenc, benc2, W1, W2, vt8)

    return (top_p.reshape(B), top_i.reshape(B))


# speedup vs baseline: 3.4220x; 2.4203x over previous
import jax
import jax.numpy as jnp
import numpy as np
from jax.experimental import pallas as pl

B, S, H = 128, 8192, 128
NSUB = 8
CS = S // NSUB
NEG = float(np.log(np.float32(1e-45)))


def _dotg(a, b, dims):
    return jax.lax.dot_general(a.astype(jnp.bfloat16), b.astype(jnp.bfloat16),
                               (dims, ((), ())),
                               preferred_element_type=jnp.float32)


def _attn_kernel(mask_ref, userc_ref, x4_ref, wenc_ref, benc_ref,
                 w1_ref, w2_ref, vt_ref, out_p_ref, out_i_ref):
    f32 = jnp.float32
    b = pl.program_id(0)

    wenc = wenc_ref[...]
    bencT = benc_ref[...]
    w1 = w1_ref[...]
    vt8 = vt_ref[...]
    dtT = _dotg(w2_ref[...], userc_ref[0],
                ((1,), (0,)))[:, 0:1]

    us = []
    for j in range(NSUB):
        x4_j = x4_ref[0][:, j * CS:(j + 1) * CS]
        encT = _dotg(wenc, x4_j, ((1,), (0,))) + bencT
        etT = _dotg(w1, encT, ((1,), (0,)))
        tT = jnp.tanh(etT + dtT)
        us.append(_dotg(vt8, tT, ((1,), (0,)))[0:1, :])

    score = jnp.where(mask_ref[0] != 0,
                      jnp.concatenate(us, axis=0), f32(NEG)) * f32(10.0)

    m = jnp.max(score)
    se = jnp.sum(jnp.exp(score - m))
    iota = (jax.lax.broadcasted_iota(jnp.int32, (NSUB, CS), 0) * CS
            + jax.lax.broadcasted_iota(jnp.int32, (NSUB, CS), 1))
    ix = jnp.min(jnp.where(score == m, iota, jnp.int32(S)))

    rowsel = jax.lax.broadcasted_iota(jnp.int32, (8, 1), 0) == (b % 8)
    out_p_ref[...] = jnp.where(rowsel, f32(1.0) / se, out_p_ref[...])
    out_i_ref[...] = jnp.where(rowsel, ix, out_i_ref[...])


@jax.jit
def kernel(mask, user, static_server_seq, tmp_server_capacity, server_active,
           Wenc, benc, W1, W2, vt):
    x4 = jnp.stack([static_server_seq[:, :, 0], static_server_seq[:, :, 1],
                    tmp_server_capacity[:, :, 0], server_active[:, :, 0]],
                   axis=1)
    mask3 = mask.reshape(B, NSUB, CS).astype(jnp.float32)
    userc8 = jnp.broadcast_to(user.reshape(B, H, 1), (B, H, 8))
    bencT = benc.reshape(H, 1)
    vt8 = jnp.broadcast_to(vt, (8, H))

    row3 = lambda b: (b, 0, 0)
    rep2 = lambda b: (0, 0)

    top_p, top_i = pl.pallas_call(
        _attn_kernel,
        grid=(B,),
        in_specs=[
            pl.BlockSpec((1, NSUB, CS), row3),
            pl.BlockSpec((1, H, 8), row3),
            pl.BlockSpec((1, 4, S), row3),
            pl.BlockSpec((H, 4), rep2),
            pl.BlockSpec((H, 1), rep2),
            pl.BlockSpec((H, H), rep2),
            pl.BlockSpec((H, H), rep2),
            pl.BlockSpec((8, H), rep2),
        ],
        out_specs=[
            pl.BlockSpec((8, 1), lambda b: (b // 8, 0)),
            pl.BlockSpec((8, 1), lambda b: (b // 8, 0)),
        ],
        out_shape=[
            jax.ShapeDtypeStruct((B, 1), jnp.float32),
            jax.ShapeDtypeStruct((B, 1), jnp.int32),
        ],
    )(mask3, userc8, x4, Wenc, bencT, W1, W2, vt8)

    return (top_p.reshape(B), top_i.reshape(B))


# speedup vs baseline: 6.1493x; 1.7970x over previous
import jax
import jax.numpy as jnp
import numpy as np
from jax.experimental import pallas as pl

B, S, H = 128, 8192, 128
NSUB = 4
CS = S // NSUB
NEG = float(np.log(np.float32(1e-45)))


def _dotg(a, b, dims):
    return jax.lax.dot_general(a.astype(jnp.bfloat16), b.astype(jnp.bfloat16),
                               (dims, ((), ())),
                               preferred_element_type=jnp.float32)


def _attn_kernel(mask_ref, userc_ref, x4_ref, wenc_ref, benc_ref,
                 w1_ref, w2_ref, vt_ref, out_p_ref, out_i_ref):
    f32 = jnp.float32
    b = pl.program_id(0)

    wenc = wenc_ref[...]
    bencT = benc_ref[...]
    w1 = w1_ref[...]
    vt8 = vt_ref[...]
    dtT = _dotg(w2_ref[...], userc_ref[0],
                ((1,), (0,)))[:, 0:1]

    encs = []
    for j in range(NSUB):
        x4_j = x4_ref[0][:, j * CS:(j + 1) * CS]
        encs.append(_dotg(wenc, x4_j, ((1,), (0,))) + bencT)
    ts = [jnp.tanh(_dotg(w1, e, ((1,), (0,))) + dtT) for e in encs]
    us = [_dotg(vt8, t, ((1,), (0,)))[0:1, :] for t in ts]

    score = jnp.where(mask_ref[0] != 0,
                      jnp.concatenate(us, axis=0), f32(NEG)) * f32(10.0)

    m = jnp.max(score)
    se = jnp.sum(jnp.exp(score - m))
    iota = (jax.lax.broadcasted_iota(jnp.int32, (NSUB, CS), 0) * CS
            + jax.lax.broadcasted_iota(jnp.int32, (NSUB, CS), 1))
    ix = jnp.min(jnp.where(score == m, iota, jnp.int32(S)))

    rowsel = jax.lax.broadcasted_iota(jnp.int32, (8, 1), 0) == (b % 8)
    out_p_ref[...] = jnp.where(rowsel, f32(1.0) / se, out_p_ref[...])
    out_i_ref[...] = jnp.where(rowsel, ix, out_i_ref[...])


@jax.jit
def kernel(mask, user, static_server_seq, tmp_server_capacity, server_active,
           Wenc, benc, W1, W2, vt):
    x4 = jnp.stack([static_server_seq[:, :, 0], static_server_seq[:, :, 1],
                    tmp_server_capacity[:, :, 0], server_active[:, :, 0]],
                   axis=1)
    mask3 = mask.reshape(B, NSUB, CS).astype(jnp.float32)
    userc8 = jnp.broadcast_to(user.reshape(B, H, 1), (B, H, 8))
    bencT = benc.reshape(H, 1)
    vt8 = jnp.broadcast_to(vt, (8, H))

    row3 = lambda b: (b, 0, 0)
    rep2 = lambda b: (0, 0)

    top_p, top_i = pl.pallas_call(
        _attn_kernel,
        grid=(B,),
        in_specs=[
            pl.BlockSpec((1, NSUB, CS), row3),
            pl.BlockSpec((1, H, 8), row3),
            pl.BlockSpec((1, 4, S), row3),
            pl.BlockSpec((H, 4), rep2),
            pl.BlockSpec((H, 1), rep2),
            pl.BlockSpec((H, H), rep2),
            pl.BlockSpec((H, H), rep2),
            pl.BlockSpec((8, H), rep2),
        ],
        out_specs=[
            pl.BlockSpec((8, 1), lambda b: (b // 8, 0)),
            pl.BlockSpec((8, 1), lambda b: (b // 8, 0)),
        ],
        out_shape=[
            jax.ShapeDtypeStruct((B, 1), jnp.float32),
            jax.ShapeDtypeStruct((B, 1), jnp.int32),
        ],
    )(mask3, userc8, x4, Wenc, bencT, W1, W2, vt8)

    return (top_p.reshape(B), top_i.reshape(B))
